# baseline (speedup 1.0000x reference)
import functools
from math import sqrt

import jax
import jax.numpy as jnp
from jax import lax
from jax.experimental import pallas as pl
from jax.experimental.pallas import tpu as pltpu
from jax.experimental.pallas import tpu_sc as plsc

_MODEL_DIM = 64
_SCALE = sqrt(_MODEL_DIM)


def _make_sc_lookup(vocab, dim, n_seq, n_batch):
    info = plsc.get_sparse_core_info()
    nc, ns, lanes = info.num_cores, info.num_subcores, info.num_lanes
    nw = nc * ns
    assert n_batch % nw == 0
    slab = n_batch // nw
    ch = slab // 2
    n_chunks = 2 * n_seq
    mesh = plsc.VectorSubcoreMesh(core_axis_name="c", subcore_axis_name="s")

    @functools.partial(
        pl.kernel,
        mesh=mesh,
        compiler_params=pltpu.CompilerParams(use_tc_tiling_on_sc=True),
        out_type=jax.ShapeDtypeStruct((n_seq, dim, n_batch), jnp.float32),
        scratch_types=[
            pltpu.VMEM((ch,), jnp.int32),
            pltpu.VMEM((ch,), jnp.int32),
            pltpu.VMEM((ch, 2 * dim), jnp.float32),
            pltpu.VMEM((ch, 2 * dim), jnp.float32),
            pltpu.VMEM((dim, ch), jnp.float32),
            pltpu.SemaphoreType.DMA,
            pltpu.SemaphoreType.DMA,
            pltpu.SemaphoreType.DMA,
            pltpu.SemaphoreType.DMA,
        ],
    )
    def k(idx_hbm, table_hbm, out_hbm, i0, i1, a0, a1, dummy, g0, g1, o0, o1):
        wid = lax.axis_index("s") * nc + lax.axis_index("c")
        base = wid * slab
        ibuf, abuf = (i0, i1), (a0, a1)
        gs, os = (g0, g1), (o0, o1)

        def idx_copy(c, b):
            pltpu.sync_copy(
                idx_hbm.at[c // 2, pl.ds(base + (c % 2) * ch, ch)], ibuf[b]
            )

        def gather(b):
            pltpu.async_copy(table_hbm.at[ibuf[b]], abuf[b], gs[b])

        def gwait(b):
            pltpu.make_async_copy(table_hbm.at[ibuf[b]], abuf[b], gs[b]).wait()

        def scale(b):
            r = abuf[b]

            def srow(row, cc):
                for q in range(dim // lanes):
                    sl = pl.ds(q * lanes, lanes)
                    r[row, sl] = r[row, sl] * _SCALE
                return cc

            lax.fori_loop(0, ch, srow, 0, unroll=4)

        def fire_writes(c, b):
            t = c // 2
            off = base + (c % 2) * ch

            def wd(d, cc):
                pltpu.async_copy(
                    abuf[b].at[:, d], out_hbm.at[t, d, pl.ds(off, ch)], os[b]
                )
                return cc

            lax.fori_loop(0, dim, wd, 0, unroll=8)

        def drain(c, b):
            t = c // 2
            off = base + (c % 2) * ch
            pltpu.make_async_copy(
                out_hbm.at[t, :, pl.ds(off, ch)], dummy, os[b]
            ).wait()

        idx_copy(0, 0)
        gather(0)

        def body(g, carry):
            c0 = 2 * g
            c1 = c0 + 1
            idx_copy(c1, 1)

            @pl.when(g > 0)
            def _():
                drain(c1 - 2, 1)

            gather(1)
            gwait(0)
            scale(0)
            fire_writes(c0, 0)

            @pl.when(c1 + 1 < n_chunks)
            def _():
                idx_copy(c1 + 1, 0)

            drain(c0, 0)

            @pl.when(c1 + 1 < n_chunks)
            def _():
                gather(0)

            gwait(1)
            scale(1)
            fire_writes(c1, 1)
            return carry

        lax.fori_loop(0, n_chunks // 2, body, 0)
        drain(n_chunks - 1, 1)

    return k


def kernel(inputs, weight):
    b, t = inputs.shape
    vocab, dim = weight.shape
    idx_t = inputs.T.astype(jnp.int32)
    table128 = jnp.tile(weight, (1, 2))
    lookup = _make_sc_lookup(vocab, dim, t, b)
    out_t = lookup(idx_t, table128)
    return out_t.transpose(2, 0, 1)


# speedup vs baseline: 58.4079x; 58.4079x over previous
import functools
from math import sqrt

import jax
import jax.numpy as jnp
from jax import lax
from jax.experimental import pallas as pl
from jax.experimental.pallas import tpu as pltpu
from jax.experimental.pallas import tpu_sc as plsc

_MODEL_DIM = 64
_SCALE = sqrt(_MODEL_DIM)


def _make_sc_lookup(vocab, dim, n_seq, n_batch):
    info = plsc.get_sparse_core_info()
    nc, ns, lanes = info.num_cores, info.num_subcores, info.num_lanes
    nw = nc * ns
    assert n_batch % nw == 0
    slab = n_batch // nw
    ch = slab // 2
    n_chunks = 2 * n_seq
    mesh = plsc.VectorSubcoreMesh(core_axis_name="c", subcore_axis_name="s")

    @functools.partial(
        pl.kernel,
        mesh=mesh,
        compiler_params=pltpu.CompilerParams(
            use_tc_tiling_on_sc=True, needs_layout_passes=False
        ),
        out_type=jax.ShapeDtypeStruct((n_seq, dim, n_batch), jnp.float32),
        scratch_types=[
            pltpu.VMEM((ch,), jnp.int32),
            pltpu.VMEM((ch,), jnp.int32),
            pltpu.VMEM((ch,), jnp.int32),
            pltpu.VMEM((ch,), jnp.int32),
            pltpu.VMEM((ch, 2 * dim), jnp.float32),
            pltpu.VMEM((ch, 2 * dim), jnp.float32),
            pltpu.VMEM((dim, ch), jnp.float32),
            pltpu.VMEM((dim, ch), jnp.float32),
            pltpu.SemaphoreType.DMA,
            pltpu.SemaphoreType.DMA,
            pltpu.SemaphoreType.DMA,
            pltpu.SemaphoreType.DMA,
        ],
    )
    def k(idx_hbm, table_hbm, out_hbm,
          i0, i1, p0, p1, a0, a1, t0, t1, g0, g1, o0, o1):
        wid = lax.axis_index("s") * nc + lax.axis_index("c")
        base = wid * slab
        ibuf, pbuf = (i0, i1), (p0, p1)
        abuf, tbuf = (a0, a1), (t0, t1)
        gs, os = (g0, g1), (o0, o1)
        lane_iota = lax.iota(jnp.int32, lanes)

        def idx_stage(c, b):
            pltpu.sync_copy(
                idx_hbm.at[c // 2, pl.ds(base + (c % 2) * ch, ch)], ibuf[b]
            )

            def prow(j, cc):
                sl = pl.ds(j * lanes, lanes)
                pbuf[b][sl] = lax.shift_right_logical(ibuf[b][sl], 1)
                return cc

            lax.fori_loop(0, ch // lanes, prow, 0, unroll=4)

        def gather(b):
            pltpu.async_copy(table_hbm.at[pbuf[b]], abuf[b], gs[b])

        def gwait(b):
            pltpu.make_async_copy(table_hbm.at[pbuf[b]], abuf[b], gs[b]).wait()

        def transpose_scale(b):
            rows, blk = abuf[b], tbuf[b]

            def bcol(s, cc):
                bsl = pl.ds(s * lanes, lanes)
                row16 = s * lanes + lane_iota
                col0 = (ibuf[b][bsl] & 1) * dim

                def dloop(d, c3):
                    vals = plsc.load_gather(rows, [row16, col0 + d])
                    blk[d, bsl] = vals * _SCALE
                    return c3

                lax.fori_loop(0, dim, dloop, 0, unroll=8)
                return cc

            lax.fori_loop(0, ch // lanes, bcol, 0)

        def out_start(c, b):
            pltpu.async_copy(
                tbuf[b],
                out_hbm.at[c // 2, :, pl.ds(base + (c % 2) * ch, ch)],
                os[b],
            )

        def out_wait(b):
            pltpu.make_async_copy(
                tbuf[b], out_hbm.at[0, :, pl.ds(base, ch)], os[b]
            ).wait()

        idx_stage(0, 0)
        gather(0)

        def body(g, carry):
            c0 = 2 * g
            c1 = c0 + 1
            idx_stage(c1, 1)
            gather(1)
            gwait(0)

            @pl.when(g > 0)
            def _():
                out_wait(0)

            transpose_scale(0)
            out_start(c0, 0)

            @pl.when(c1 + 1 < n_chunks)
            def _():
                idx_stage(c1 + 1, 0)
                gather(0)

            gwait(1)

            @pl.when(g > 0)
            def _():
                out_wait(1)

            transpose_scale(1)
            out_start(c1, 1)
            return carry

        lax.fori_loop(0, n_chunks // 2, body, 0)
        out_wait(0)
        out_wait(1)

    return k


def kernel(inputs, weight):
    b, t = inputs.shape
    vocab, dim = weight.shape
    idx_t = inputs.T.astype(jnp.int32)
    table2 = weight.reshape(vocab // 2, 2 * dim)
    lookup = _make_sc_lookup(vocab, dim, t, b)
    out_t = lookup(idx_t, table2)
    return out_t.transpose(2, 0, 1)


# speedup vs baseline: 88.3192x; 1.5121x over previous
import functools
from math import sqrt

import jax
import jax.numpy as jnp
from jax import lax
from jax.experimental import pallas as pl
from jax.experimental.pallas import tpu as pltpu
from jax.experimental.pallas import tpu_sc as plsc

_MODEL_DIM = 64
_SCALE = sqrt(_MODEL_DIM)


def _make_sc_lookup(vocab, dim, n_seq, n_batch):
    info = plsc.get_sparse_core_info()
    nc, ns, lanes = info.num_cores, info.num_subcores, info.num_lanes
    nw = nc * ns
    assert n_batch % nw == 0
    slab = n_batch // nw
    ch = slab // 2
    n_chunks = 2 * n_seq
    mesh = plsc.VectorSubcoreMesh(core_axis_name="c", subcore_axis_name="s")

    @functools.partial(
        pl.kernel,
        mesh=mesh,
        compiler_params=pltpu.CompilerParams(
            use_tc_tiling_on_sc=True, needs_layout_passes=False
        ),
        out_type=jax.ShapeDtypeStruct((n_seq, dim, n_batch), jnp.float32),
        scratch_types=[
            pltpu.VMEM((ch,), jnp.int32),
            pltpu.VMEM((ch,), jnp.int32),
            pltpu.VMEM((ch,), jnp.int32),
            pltpu.VMEM((ch,), jnp.int32),
            pltpu.VMEM((ch, 2 * dim), jnp.float32),
            pltpu.VMEM((ch, 2 * dim), jnp.float32),
            pltpu.VMEM((dim, ch), jnp.float32),
            pltpu.VMEM((dim, ch), jnp.float32),
            pltpu.SemaphoreType.DMA,
            pltpu.SemaphoreType.DMA,
            pltpu.SemaphoreType.DMA,
            pltpu.SemaphoreType.DMA,
        ],
    )
    def k(idx_hbm, table_hbm, out_hbm,
          i0, i1, p0, p1, a0, a1, t0, t1, g0, g1, o0, o1):
        wid = lax.axis_index("s") * nc + lax.axis_index("c")
        base = wid * slab
        ibuf, pbuf = (i0, i1), (p0, p1)
        abuf, tbuf = (a0, a1), (t0, t1)
        gs, os = (g0, g1), (o0, o1)
        lane_iota = lax.iota(jnp.int32, lanes)

        def idx_stage(c, b):
            pltpu.sync_copy(
                idx_hbm.at[c // 2, pl.ds(base + (c % 2) * ch, ch)], ibuf[b]
            )

            def prow(j, cc):
                sl = pl.ds(j * lanes, lanes)
                pbuf[b][sl] = lax.shift_right_logical(ibuf[b][sl], 1)
                return cc

            lax.fori_loop(0, ch // lanes, prow, 0, unroll=4)

        def gather(b):
            pltpu.async_copy(table_hbm.at[pbuf[b]], abuf[b], gs[b])

        def gwait(b):
            pltpu.make_async_copy(table_hbm.at[pbuf[b]], abuf[b], gs[b]).wait()

        def transpose_scale(b):
            rows, blk = abuf[b], tbuf[b]

            def bcol(s, cc):
                bsl = pl.ds(s * lanes, lanes)
                row16 = s * lanes + lane_iota
                col0 = (ibuf[b][bsl] & 1) * dim

                @plsc.parallel_loop(0, dim, step=8)
                def dloop(d):
                    vals = [
                        plsc.load_gather(rows, [row16, col0 + (d + u)])
                        for u in range(8)
                    ]
                    for u in range(8):
                        blk[d + u, bsl] = vals[u] * _SCALE

                return cc

            lax.fori_loop(0, ch // lanes, bcol, 0)

        def out_start(c, b):
            pltpu.async_copy(
                tbuf[b],
                out_hbm.at[c // 2, :, pl.ds(base + (c % 2) * ch, ch)],
                os[b],
            )

        def out_wait(b):
            pltpu.make_async_copy(
                tbuf[b], out_hbm.at[0, :, pl.ds(base, ch)], os[b]
            ).wait()

        idx_stage(0, 0)
        gather(0)

        def body(g, carry):
            c0 = 2 * g
            c1 = c0 + 1
            idx_stage(c1, 1)
            gather(1)
            gwait(0)

            @pl.when(g > 0)
            def _():
                out_wait(0)

            transpose_scale(0)
            out_start(c0, 0)

            @pl.when(c1 + 1 < n_chunks)
            def _():
                idx_stage(c1 + 1, 0)
                gather(0)

            gwait(1)

            @pl.when(g > 0)
            def _():
                out_wait(1)

            transpose_scale(1)
            out_start(c1, 1)
            return carry

        lax.fori_loop(0, n_chunks // 2, body, 0)
        out_wait(0)
        out_wait(1)

    return k


def kernel(inputs, weight):
    b, t = inputs.shape
    vocab, dim = weight.shape
    idx_t = inputs.T.astype(jnp.int32)
    table2 = weight.reshape(vocab // 2, 2 * dim)
    lookup = _make_sc_lookup(vocab, dim, t, b)
    out_t = lookup(idx_t, table2)
    return out_t.transpose(2, 0, 1)
